# baseline (speedup 1.0000x reference)
import functools

import jax
import jax.numpy as jnp
from jax import lax
from jax.experimental import pallas as pl
from jax.experimental.pallas import tpu as pltpu
from jax.experimental.pallas import tpu_sc as plsc

MU_CONST = 3.5
BATCH_N = 16384
KDIM = 64
NUM_CORES = 2
NUM_SUBCORES = 16
NW = NUM_CORES * NUM_SUBCORES
BW = BATCH_N // NW
CHUNK = 128
NCHUNK = BW // CHUNK
L = 16

_mesh = plsc.VectorSubcoreMesh(
    core_axis_name="c", subcore_axis_name="s",
    num_cores=NUM_CORES, num_subcores=NUM_SUBCORES)


@functools.partial(
    pl.kernel,
    out_type=jax.ShapeDtypeStruct((BATCH_N,), jnp.float32),
    mesh=_mesh,
    compiler_params=pltpu.CompilerParams(
        use_tc_tiling_on_sc=False, needs_layout_passes=False),
    scratch_types=[
        pltpu.VMEM((NCHUNK, CHUNK), jnp.int32),
        pltpu.VMEM((NCHUNK, CHUNK), jnp.int32),
        pltpu.VMEM((BW, KDIM), jnp.float32),
        pltpu.VMEM((BW, KDIM), jnp.float32),
        pltpu.VMEM((BW,), jnp.float32),
        pltpu.VMEM((BW,), jnp.float32),
        pltpu.VMEM((BW,), jnp.float32),
        pltpu.SemaphoreType.DMA,
    ],
)
def _svdpp_sc(uidx_hbm, iidx_hbm, uemb_hbm, iemb_hbm, ubias_hbm, ibias_hbm,
              out_hbm, uidx_v, iidx_v, p_v, q_v, bu_v, bi_v, out_v, sem):
    wid = lax.axis_index("s") * NUM_CORES + lax.axis_index("c")
    base = wid * BW

    pltpu.sync_copy(uidx_hbm.at[pl.ds(wid * NCHUNK, NCHUNK)], uidx_v)
    pltpu.sync_copy(iidx_hbm.at[pl.ds(wid * NCHUNK, NCHUNK)], iidx_v)

    copies = []
    for c in range(NCHUNK):
        rows = pl.ds(c * CHUNK, CHUNK)
        copies.append(pltpu.async_copy(
            uemb_hbm.at[uidx_v.at[c]], p_v.at[rows], sem))
        copies.append(pltpu.async_copy(
            iemb_hbm.at[iidx_v.at[c]], q_v.at[rows], sem))
        copies.append(pltpu.async_copy(
            ubias_hbm.at[uidx_v.at[c]], bu_v.at[rows], sem))
        copies.append(pltpu.async_copy(
            ibias_hbm.at[iidx_v.at[c]], bi_v.at[rows], sem))
    for cp in copies:
        cp.wait()

    lane = lax.iota(jnp.int32, L)

    def block_body(b, _):
        rows = b * L + lane

        def col_body(k, acc):
            cols = (lane + k) & (KDIM - 1)
            pu = plsc.load_gather(p_v, [rows, cols])
            qu = plsc.load_gather(q_v, [rows, cols])
            return acc + pu * qu

        acc = lax.fori_loop(0, KDIM, col_body,
                            jnp.zeros((L,), jnp.float32), unroll=8)
        out_v[pl.ds(b * L, L)] = acc
        return ()

    lax.fori_loop(0, BW // L, block_body, ())

    for v in range(BW // L):
        s = pl.ds(v * L, L)
        out_v[s] = out_v[s] + bu_v[s] + bi_v[s] + MU_CONST

    pltpu.sync_copy(out_v, out_hbm.at[pl.ds(base, BW)])


def kernel(user_input, item_input, user_emb, item_emb, user_bias_tab,
           item_bias_tab):
    uidx = user_input.reshape(NW * NCHUNK, CHUNK).astype(jnp.int32)
    iidx = item_input.reshape(NW * NCHUNK, CHUNK).astype(jnp.int32)
    out = _svdpp_sc(uidx, iidx, user_emb, item_emb,
                    user_bias_tab.reshape(-1), item_bias_tab.reshape(-1))
    return out.reshape(BATCH_N, 1)


# speedup vs baseline: 1.0244x; 1.0244x over previous
import functools

import jax
import jax.numpy as jnp
from jax import lax
from jax.experimental import pallas as pl
from jax.experimental.pallas import tpu as pltpu
from jax.experimental.pallas import tpu_sc as plsc

MU_CONST = 3.5
BATCH_N = 16384
KDIM = 64
NUM_CORES = 2
NUM_SUBCORES = 16
NW = NUM_CORES * NUM_SUBCORES
BW = BATCH_N // NW
L = 16
HB = 128
NPASS = BW // HB
NGROUP = HB // L

_mesh = plsc.VectorSubcoreMesh(
    core_axis_name="c", subcore_axis_name="s",
    num_cores=NUM_CORES, num_subcores=NUM_SUBCORES)


@functools.partial(
    pl.kernel,
    out_type=jax.ShapeDtypeStruct((BATCH_N,), jnp.float32),
    mesh=_mesh,
    compiler_params=pltpu.CompilerParams(needs_layout_passes=False),
    scratch_types=[
        pltpu.VMEM((BW,), jnp.int32),
        pltpu.VMEM((BW,), jnp.int32),
        pltpu.VMEM((HB, KDIM), jnp.float32),
        pltpu.VMEM((HB, KDIM), jnp.float32),
        pltpu.VMEM((HB, 1), jnp.float32),
        pltpu.VMEM((HB, 1), jnp.float32),
        pltpu.VMEM((BW,), jnp.float32),
        pltpu.SemaphoreType.DMA,
    ],
)
def _svdpp_sc(uidx_hbm, iidx_hbm, uemb_hbm, iemb_hbm, ubias_hbm, ibias_hbm,
              out_hbm, uidx_v, iidx_v, p_v, q_v, bu_v, bi_v, out_v, sem):
    wid = lax.axis_index("s") * NUM_CORES + lax.axis_index("c")
    base = wid * BW

    pltpu.sync_copy(uidx_hbm.at[pl.ds(base, BW)], uidx_v)
    pltpu.sync_copy(iidx_hbm.at[pl.ds(base, BW)], iidx_v)

    lane = lax.iota(jnp.int32, L)
    zeros_i = jnp.zeros((L,), jnp.int32)

    def one_pass(p, _):
        poff = p * HB

        def issue(g, _):
            vec_u = uidx_v[pl.ds(poff + g * L, L)]
            vec_i = iidx_v[pl.ds(poff + g * L, L)]
            for j in range(L):
                r = g * L + j
                ru = vec_u[j]
                ri = vec_i[j]
                pltpu.async_copy(uemb_hbm.at[pl.ds(ru, 1)],
                                 p_v.at[pl.ds(r, 1)], sem)
                pltpu.async_copy(iemb_hbm.at[pl.ds(ri, 1)],
                                 q_v.at[pl.ds(r, 1)], sem)
                pltpu.async_copy(ubias_hbm.at[pl.ds(ru, 1)],
                                 bu_v.at[pl.ds(r, 1)], sem)
                pltpu.async_copy(ibias_hbm.at[pl.ds(ri, 1)],
                                 bi_v.at[pl.ds(r, 1)], sem)
            return ()

        lax.fori_loop(0, NGROUP, issue, ())

        pltpu.make_async_copy(uemb_hbm.at[pl.ds(0, HB)], p_v, sem).wait()
        pltpu.make_async_copy(iemb_hbm.at[pl.ds(0, HB)], q_v, sem).wait()
        pltpu.make_async_copy(ubias_hbm.at[pl.ds(0, HB)], bu_v, sem).wait()
        pltpu.make_async_copy(ibias_hbm.at[pl.ds(0, HB)], bi_v, sem).wait()

        def block_body(g, _):
            rows = g * L + lane

            def col_body(k, acc):
                cols = (lane + k) & (KDIM - 1)
                pu = plsc.load_gather(p_v, [rows, cols])
                qu = plsc.load_gather(q_v, [rows, cols])
                return acc + pu * qu

            acc = lax.fori_loop(0, KDIM, col_body,
                                jnp.zeros((L,), jnp.float32), unroll=8)
            bu = plsc.load_gather(bu_v, [rows, zeros_i])
            bi = plsc.load_gather(bi_v, [rows, zeros_i])
            out_v[pl.ds(poff + g * L, L)] = acc + bu + bi + MU_CONST
            return ()

        lax.fori_loop(0, NGROUP, block_body, ())
        return ()

    lax.fori_loop(0, NPASS, one_pass, ())

    pltpu.sync_copy(out_v, out_hbm.at[pl.ds(base, BW)])


def kernel(user_input, item_input, user_emb, item_emb, user_bias_tab,
           item_bias_tab):
    uidx = user_input.reshape(BATCH_N).astype(jnp.int32)
    iidx = item_input.reshape(BATCH_N).astype(jnp.int32)
    out = _svdpp_sc(uidx, iidx, user_emb, item_emb,
                    user_bias_tab, item_bias_tab)
    return out.reshape(BATCH_N, 1)
